# Initial kernel scaffold; baseline (speedup 1.0000x reference)
import functools

import jax
import jax.numpy as jnp
from jax import lax
from jax.experimental import pallas as pl
from jax.experimental.pallas import tpu as pltpu
from jax.experimental.pallas import tpu_sc as plsc

N = 10000
E = 320000
D_IN = 128
D_HID = 128
N_CLASSES = 16

NPAD = 10240
EPAD = 327680
EROWS = EPAD // 128
ROWS_PER_TILE = EROWS // 32
CHUNK_ROWS = 4
N_STEPS = ROWS_PER_TILE // CHUNK_ROWS
NODES_PER_TILE = NPAD // 16

_MESH = plsc.VectorSubcoreMesh(core_axis_name="c", subcore_axis_name="s")


@functools.partial(
    pl.kernel,
    mesh=_MESH,
    out_type=jax.ShapeDtypeStruct((2, NPAD), jnp.float32),
    scratch_types=[
        pltpu.VMEM((CHUNK_ROWS, 128), jnp.int32),
        pltpu.VMEM((128,), jnp.float32),
        pltpu.VMEM((NODES_PER_TILE,), jnp.float32),
        pltpu.VMEM_SHARED((NPAD,), jnp.float32),
        pltpu.SemaphoreType.DMA,
    ],
)
def _deg_kernel(zeros_hbm, dst_hbm, out_hbm, didx, ones, stage, acc, sem):
    cid = lax.axis_index("c")
    sid = lax.axis_index("s")
    for j in range(8):
        ones[pl.ds(j * 16, 16)] = jnp.ones((16,), jnp.float32)
    base_n = sid * NODES_PER_TILE
    pltpu.sync_copy(zeros_hbm, stage)
    pltpu.sync_copy(stage, acc.at[pl.ds(base_n, NODES_PER_TILE)])
    plsc.subcore_barrier()

    row0 = (cid * 16 + sid) * ROWS_PER_TILE

    def step(k, carry):
        r = row0 + k * CHUNK_ROWS
        pltpu.sync_copy(dst_hbm.at[pl.ds(r, CHUNK_ROWS)], didx)
        cps = [
            pltpu.async_copy(ones, acc.at[didx.at[j]], sem, add=True)
            for j in range(CHUNK_ROWS)
        ]
        for c in cps:
            c.wait()
        return carry

    lax.fori_loop(0, N_STEPS, step, 0)
    plsc.subcore_barrier()
    pltpu.sync_copy(acc.at[pl.ds(base_n, NODES_PER_TILE)], stage)
    pltpu.sync_copy(stage, out_hbm.at[cid, pl.ds(base_n, NODES_PER_TILE)])


def _make_agg_kernel(d: int):
    chunk_e = CHUNK_ROWS * 128

    @functools.partial(
        pl.kernel,
        mesh=_MESH,
        out_type=jax.ShapeDtypeStruct((2, NPAD, d), jnp.float32),
        scratch_types=[
            pltpu.VMEM((CHUNK_ROWS, 128), jnp.int32),
            pltpu.VMEM((CHUNK_ROWS, 128), jnp.int32),
            pltpu.VMEM((chunk_e, d), jnp.float32),
            pltpu.VMEM_SHARED((NPAD, d), jnp.float32),
            pltpu.SemaphoreType.DMA,
            pltpu.SemaphoreType.DMA,
        ],
    )
    def agg_kernel(zeros_hbm, src_hbm, dst_hbm, g_hbm, out_hbm,
                   sidx, didx, rows, acc, gsem, ssem):
        cid = lax.axis_index("c")
        sid = lax.axis_index("s")
        base_n = sid * NODES_PER_TILE
        pltpu.sync_copy(zeros_hbm, rows)
        pltpu.sync_copy(rows, acc.at[pl.ds(base_n, chunk_e)])
        pltpu.sync_copy(rows.at[pl.ds(0, NODES_PER_TILE - chunk_e)],
                        acc.at[pl.ds(base_n + chunk_e, NODES_PER_TILE - chunk_e)])
        plsc.subcore_barrier()

        row0 = (cid * 16 + sid) * ROWS_PER_TILE

        def step(k, carry):
            r = row0 + k * CHUNK_ROWS
            pltpu.sync_copy(src_hbm.at[pl.ds(r, CHUNK_ROWS)], sidx)
            pltpu.sync_copy(dst_hbm.at[pl.ds(r, CHUNK_ROWS)], didx)
            gs = [
                pltpu.async_copy(g_hbm.at[sidx.at[j]],
                                 rows.at[pl.ds(j * 128, 128)], gsem)
                for j in range(CHUNK_ROWS)
            ]
            for c in gs:
                c.wait()
            ss = [
                pltpu.async_copy(rows.at[pl.ds(j * 128, 128)],
                                 acc.at[didx.at[j]], ssem, add=True)
                for j in range(CHUNK_ROWS)
            ]
            for c in ss:
                c.wait()
            return carry

        lax.fori_loop(0, N_STEPS, step, 0)
        plsc.subcore_barrier()
        pltpu.sync_copy(acc.at[pl.ds(base_n, chunk_e)], rows)
        pltpu.sync_copy(rows, out_hbm.at[cid, pl.ds(base_n, chunk_e)])
        pltpu.sync_copy(acc.at[pl.ds(base_n + chunk_e, NODES_PER_TILE - chunk_e)],
                        rows.at[pl.ds(0, NODES_PER_TILE - chunk_e)])
        pltpu.sync_copy(rows.at[pl.ds(0, NODES_PER_TILE - chunk_e)],
                        out_hbm.at[cid, pl.ds(base_n + chunk_e, NODES_PER_TILE - chunk_e)])

    return agg_kernel


_agg_128 = _make_agg_kernel(D_HID)
_agg_16 = _make_agg_kernel(N_CLASSES)


RB = 1024
GRID = NPAD // RB


def _dinv_body(deg_ref, dinv_ref):
    deg = deg_ref[0, :] + deg_ref[1, :] + 1.0
    dinv_ref[...] = lax.rsqrt(deg)[None, :]


def _dinv_tc(degp):
    return pl.pallas_call(
        _dinv_body,
        grid=(1,),
        in_specs=[pl.BlockSpec((2, NPAD), lambda i: (0, 0))],
        out_specs=pl.BlockSpec((1, NPAD), lambda i: (0, 0)),
        out_shape=jax.ShapeDtypeStruct((1, NPAD), jnp.float32),
    )(degp)


def _g1_body(x_ref, w_ref, dinv_ref, g_ref):
    h = jnp.dot(x_ref[...], w_ref[...], preferred_element_type=jnp.float32)
    g_ref[...] = h * dinv_ref[...]


def _g1_tc(x_pad, w1, dinv_col):
    return pl.pallas_call(
        _g1_body,
        grid=(GRID,),
        in_specs=[
            pl.BlockSpec((RB, D_IN), lambda i: (i, 0)),
            pl.BlockSpec((D_IN, D_HID), lambda i: (0, 0)),
            pl.BlockSpec((RB, 1), lambda i: (i, 0)),
        ],
        out_specs=pl.BlockSpec((RB, D_HID), lambda i: (i, 0)),
        out_shape=jax.ShapeDtypeStruct((NPAD, D_HID), jnp.float32),
    )(x_pad, w1, dinv_col)


def _g2_body(a1_ref, g1_ref, dinv_ref, b1_ref, w2_ref, g2_ref):
    a = a1_ref[...]
    s = a[0] + a[1] + g1_ref[...]
    pre = s * dinv_ref[...] + b1_ref[...]
    h = jnp.maximum(pre, 0.0)
    g2_ref[...] = (
        jnp.dot(h, w2_ref[...], preferred_element_type=jnp.float32)
        * dinv_ref[...]
    )


def _g2_tc(a1p, g1, dinv_col, b1r, w2):
    return pl.pallas_call(
        _g2_body,
        grid=(GRID,),
        in_specs=[
            pl.BlockSpec((2, RB, D_HID), lambda i: (0, i, 0)),
            pl.BlockSpec((RB, D_HID), lambda i: (i, 0)),
            pl.BlockSpec((RB, 1), lambda i: (i, 0)),
            pl.BlockSpec((1, D_HID), lambda i: (0, 0)),
            pl.BlockSpec((D_HID, N_CLASSES), lambda i: (0, 0)),
        ],
        out_specs=pl.BlockSpec((RB, N_CLASSES), lambda i: (i, 0)),
        out_shape=jax.ShapeDtypeStruct((NPAD, N_CLASSES), jnp.float32),
    )(a1p, g1, dinv_col, b1r, w2)


def _out_body(a2_ref, g2_ref, dinv_ref, b2_ref, o_ref):
    a = a2_ref[...]
    s = a[0] + a[1] + g2_ref[...]
    o_ref[...] = s * dinv_ref[...] + b2_ref[...]


def _out_tc(a2p, g2, dinv_col, b2r):
    return pl.pallas_call(
        _out_body,
        grid=(GRID,),
        in_specs=[
            pl.BlockSpec((2, RB, N_CLASSES), lambda i: (0, i, 0)),
            pl.BlockSpec((RB, N_CLASSES), lambda i: (i, 0)),
            pl.BlockSpec((RB, 1), lambda i: (i, 0)),
            pl.BlockSpec((1, N_CLASSES), lambda i: (0, 0)),
        ],
        out_specs=pl.BlockSpec((RB, N_CLASSES), lambda i: (i, 0)),
        out_shape=jax.ShapeDtypeStruct((NPAD, N_CLASSES), jnp.float32),
    )(a2p, g2, dinv_col, b2r)


def kernel(x, edge_index, W1, b1, W2, b2):
    ei = edge_index.astype(jnp.int32)
    pad = jnp.full((EPAD - E,), N, jnp.int32)
    src2 = jnp.concatenate([ei[0], pad]).reshape(EROWS, 128)
    dst2 = jnp.concatenate([ei[1], pad]).reshape(EROWS, 128)
    x_pad = jnp.concatenate(
        [x, jnp.zeros((NPAD - N, D_IN), jnp.float32)], axis=0)
    zeros_1d = jnp.zeros((NODES_PER_TILE,), jnp.float32)
    zeros_128 = jnp.zeros((CHUNK_ROWS * 128, D_HID), jnp.float32)
    zeros_16 = jnp.zeros((CHUNK_ROWS * 128, N_CLASSES), jnp.float32)
    b1r = b1.reshape(1, D_HID)
    b2r = b2.reshape(1, N_CLASSES)

    degp = _deg_kernel(zeros_1d, dst2)
    dinv_col = _dinv_tc(degp).reshape(NPAD, 1)
    g1 = _g1_tc(x_pad, W1, dinv_col)
    a1p = _agg_128(zeros_128, src2, dst2, g1)
    g2 = _g2_tc(a1p, g1, dinv_col, b1r, W2)
    a2p = _agg_16(zeros_16, src2, dst2, g2)
    out = _out_tc(a2p, g2, dinv_col, b2r)
    return out[:N]


# speedup vs baseline: 38.5969x; 38.5969x over previous
import functools

import jax
import jax.numpy as jnp
from jax import lax
from jax.experimental import pallas as pl
from jax.experimental.pallas import tpu as pltpu
from jax.experimental.pallas import tpu_sc as plsc

N = 10000
E = 320000
D_IN = 128
D_HID = 128
N_CLASSES = 16

NPAD = 10240
EPAD = 327680
EROWS = EPAD // 128
ROWS_PER_TILE = EROWS // 32
CHUNK_ROWS = 4
N_STEPS = ROWS_PER_TILE // CHUNK_ROWS
NODES_PER_TILE = NPAD // 16

_MESH = plsc.VectorSubcoreMesh(core_axis_name="c", subcore_axis_name="s")


@functools.partial(
    pl.kernel,
    mesh=_MESH,
    compiler_params=pltpu.CompilerParams(use_tc_tiling_on_sc=False),
    out_type=jax.ShapeDtypeStruct((2, NPAD), jnp.float32),
    scratch_types=[
        pltpu.VMEM((ROWS_PER_TILE, 128), jnp.int32),
        pltpu.VMEM((128,), jnp.float32),
        pltpu.VMEM_SHARED((NPAD,), jnp.float32),
        pltpu.SemaphoreType.DMA,
        pltpu.SemaphoreType.DMA,
    ],
)
def _deg_kernel(zeros_hbm, dst_hbm, out_hbm, didx, ones, acc, ssem0, ssem1):
    cid = lax.axis_index("c")
    sid = lax.axis_index("s")
    for j in range(8):
        ones[pl.ds(j * 16, 16)] = jnp.ones((16,), jnp.float32)
    base_n = sid * NODES_PER_TILE
    row0 = (cid * 16 + sid) * ROWS_PER_TILE
    pltpu.sync_copy(dst_hbm.at[pl.ds(row0, ROWS_PER_TILE)], didx)
    pltpu.sync_copy(zeros_hbm, acc.at[pl.ds(base_n, NODES_PER_TILE)])
    plsc.subcore_barrier()

    def fire(c, sem):
        for j in range(CHUNK_ROWS):
            pltpu.async_copy(ones, acc.at[didx.at[c * CHUNK_ROWS + j]], sem,
                             add=True)

    dummy = zeros_hbm.at[pl.ds(0, 128)]

    def drain(sem):
        for _ in range(CHUNK_ROWS):
            pltpu.make_async_copy(dummy, ones, sem).wait()

    fire(0, ssem0)

    def body(it, carry):
        a = 2 * it
        fire(a + 1, ssem1)
        drain(ssem0)
        fire(a + 2, ssem0)
        drain(ssem1)
        return carry

    lax.fori_loop(0, N_STEPS // 2 - 1, body, 0)
    fire(N_STEPS - 1, ssem1)
    drain(ssem0)
    drain(ssem1)
    plsc.subcore_barrier()
    pltpu.sync_copy(acc.at[pl.ds(base_n, NODES_PER_TILE)],
                    out_hbm.at[cid, pl.ds(base_n, NODES_PER_TILE)])


_CHUNK_E = CHUNK_ROWS * 128
_TAIL = NODES_PER_TILE - _CHUNK_E


def _make_agg(d, split_edges, macro_chunks, dtype=jnp.float32,
              chunk_rows=CHUNK_ROWS, table_spmem=False, table_per_core=False):
    rows_per_tile = EROWS // 32 if split_edges else EROWS // 16
    steps = rows_per_tile // chunk_rows
    mc = macro_chunks
    mrows = mc * chunk_rows
    pairs = steps // (2 * mc)
    assert steps == pairs * 2 * mc

    @functools.partial(
        pl.kernel,
        mesh=_MESH,
        compiler_params=pltpu.CompilerParams(use_tc_tiling_on_sc=False),
        out_type=jax.ShapeDtypeStruct((2, NPAD, d), dtype),
        scratch_types=[
            pltpu.VMEM((mrows, 128), jnp.int32),
            pltpu.VMEM((mrows, 128), jnp.int32),
            pltpu.VMEM((mrows, 128), jnp.int32),
            pltpu.VMEM((mrows, 128), jnp.int32),
            pltpu.VMEM((chunk_rows * 128, d), dtype),
            pltpu.VMEM((chunk_rows * 128, d), dtype),
            pltpu.VMEM_SHARED((NPAD, d), dtype),
            (pltpu.VMEM_SHARED((NPAD, d), dtype)
             if table_spmem else pltpu.SMEM((1,), jnp.int32)),
            pltpu.SemaphoreType.DMA,
            pltpu.SemaphoreType.DMA,
            pltpu.SemaphoreType.DMA,
            pltpu.SemaphoreType.DMA,
        ],
    )
    def agg_kernel(zeros_hbm, src_hbm, dst_hbm, g_hbm, out_hbm,
                   sidxA, didxA, sidxB, didxB, rows0, rows1, acc, tbl,
                   gsem0, gsem1, ssem0, ssem1):
        cid = lax.axis_index("c")
        sid = lax.axis_index("s")
        base_n = sid * NODES_PER_TILE
        if split_edges:
            row0 = (cid * 16 + sid) * rows_per_tile
        else:
            row0 = sid * rows_per_tile
        src_table = g_hbm.at[cid] if table_per_core else g_hbm
        table = tbl if table_spmem else src_table

        rowss = [rows0, rows1]
        gsems = [gsem0, gsem1]
        ssems = [ssem0, ssem1]
        chunk_e = chunk_rows * 128
        dummy = zeros_hbm.at[pl.ds(0, chunk_e)]

        def load_idx(sbuf, dbuf, r):
            pltpu.sync_copy(src_hbm.at[pl.ds(r, mrows)], sbuf)
            pltpu.sync_copy(dst_hbm.at[pl.ds(r, mrows)], dbuf)

        def fire_g(sbuf, loc, buf, sem):
            for j in range(chunk_rows):
                pltpu.async_copy(table.at[sbuf.at[loc * chunk_rows + j]],
                                 buf.at[pl.ds(j * 128, 128)], sem)

        def fire_s(dbuf, loc, buf, sem):
            for j in range(chunk_rows):
                pltpu.async_copy(buf.at[pl.ds(j * 128, 128)],
                                 acc.at[dbuf.at[loc * chunk_rows + j]], sem,
                                 add=True)

        def drain(sem, buf):
            pltpu.make_async_copy(dummy, buf, sem).wait()

        def pair_body(u, reload_a, fire_next):
            for i in range(2 * mc):
                p = i % 2
                in_a = i < mc
                loc = i if in_a else i - mc
                drain(gsems[p], rowss[p])
                fire_s(didxA if in_a else didxB, loc, rowss[p], ssems[p])
                drain(ssems[1 - p], rowss[1 - p])
                if i == 0:
                    load_idx(sidxB, didxB, row0 + (2 * u + 1) * mrows)
                if i == mc and reload_a:
                    load_idx(sidxA, didxA, row0 + (2 * u + 2) * mrows)
                if i < 2 * mc - 1:
                    i1 = i + 1
                    sb = sidxA if i1 < mc else sidxB
                    l1 = i1 if i1 < mc else i1 - mc
                    fire_g(sb, l1, rowss[1 - p], gsems[1 - p])
                elif fire_next:
                    fire_g(sidxA, 0, rowss[1 - p], gsems[1 - p])

        load_idx(sidxA, didxA, row0)
        pltpu.sync_copy(zeros_hbm.at[pl.ds(0, NODES_PER_TILE)],
                        acc.at[pl.ds(base_n, NODES_PER_TILE)])
        if table_spmem:
            pltpu.sync_copy(src_table.at[pl.ds(base_n, NODES_PER_TILE)],
                            tbl.at[pl.ds(base_n, NODES_PER_TILE)])
        pltpu.sync_copy(dummy, rows1)
        plsc.subcore_barrier()
        fire_g(sidxA, 0, rows0, gsem0)
        fire_s(didxA, 0, rows1, ssem1)

        def body(u, carry):
            pair_body(u, reload_a=True, fire_next=True)
            return carry

        lax.fori_loop(0, pairs - 1, body, 0)
        pair_body(pairs - 1, reload_a=False, fire_next=False)
        drain(ssems[1], rowss[1])
        plsc.subcore_barrier()
        pltpu.sync_copy(acc.at[pl.ds(base_n, NODES_PER_TILE)],
                        out_hbm.at[cid, pl.ds(base_n, NODES_PER_TILE)])

    return agg_kernel


_agg_128 = _make_agg(64, split_edges=False, macro_chunks=2,
                     dtype=jnp.bfloat16, chunk_rows=8, table_spmem=True,
                     table_per_core=True)
_agg_16 = _make_agg(N_CLASSES, split_edges=True, macro_chunks=5, chunk_rows=8,
                    table_spmem=True)


RB = 1024
GRID = NPAD // RB


def _dinv_block(degp_ref):
    d = degp_ref[...]
    return lax.rsqrt(d[0] + d[1] + 1.0)


def _g1_body(x_ref, w_ref, degp_ref, g_ref):
    h = jnp.dot(x_ref[...], w_ref[...], preferred_element_type=jnp.float32)
    g = (h * _dinv_block(degp_ref)).astype(jnp.bfloat16)
    g_ref[0] = g[:, :64]
    g_ref[1] = g[:, 64:]


def _g1_tc(x_pad, w1, degp2):
    return pl.pallas_call(
        _g1_body,
        grid=(GRID,),
        in_specs=[
            pl.BlockSpec((RB, D_IN), lambda i: (i, 0)),
            pl.BlockSpec((D_IN, D_HID), lambda i: (0, 0)),
            pl.BlockSpec((2, RB, 1), lambda i: (0, i, 0)),
        ],
        out_specs=pl.BlockSpec((2, RB, 64), lambda i: (0, i, 0)),
        out_shape=jax.ShapeDtypeStruct((2, NPAD, 64), jnp.bfloat16),
    )(x_pad, w1, degp2)


def _g2_body(a1_ref, g1_ref, degp_ref, b1_ref, w2_ref, g2_ref):
    dinv = _dinv_block(degp_ref)
    s = (a1_ref[...].astype(jnp.float32)
         + g1_ref[...].astype(jnp.float32))
    sfull = jnp.concatenate([s[0], s[1]], axis=1)
    pre = sfull * dinv + b1_ref[...]
    h = jnp.maximum(pre, 0.0)
    g2_ref[...] = (
        jnp.dot(h, w2_ref[...], preferred_element_type=jnp.float32)
        * dinv)


def _g2_tc(a1, g1, degp2, b1r, w2):
    return pl.pallas_call(
        _g2_body,
        grid=(GRID,),
        in_specs=[
            pl.BlockSpec((2, RB, 64), lambda i: (0, i, 0)),
            pl.BlockSpec((2, RB, 64), lambda i: (0, i, 0)),
            pl.BlockSpec((2, RB, 1), lambda i: (0, i, 0)),
            pl.BlockSpec((1, D_HID), lambda i: (0, 0)),
            pl.BlockSpec((D_HID, N_CLASSES), lambda i: (0, 0)),
        ],
        out_specs=pl.BlockSpec((RB, N_CLASSES), lambda i: (i, 0)),
        out_shape=jax.ShapeDtypeStruct((NPAD, N_CLASSES), jnp.float32),
    )(a1, g1, degp2, b1r, w2)


def _out_body(a2_ref, g2_ref, degp_ref, b2_ref, o_ref):
    a = a2_ref[...]
    s = a[0] + a[1] + g2_ref[...]
    o_ref[...] = s * _dinv_block(degp_ref) + b2_ref[...]


def _out_tc(a2p, g2, degp2, b2r):
    return pl.pallas_call(
        _out_body,
        grid=(GRID,),
        in_specs=[
            pl.BlockSpec((2, RB, N_CLASSES), lambda i: (0, i, 0)),
            pl.BlockSpec((RB, N_CLASSES), lambda i: (i, 0)),
            pl.BlockSpec((2, RB, 1), lambda i: (0, i, 0)),
            pl.BlockSpec((1, N_CLASSES), lambda i: (0, 0)),
        ],
        out_specs=pl.BlockSpec((RB, N_CLASSES), lambda i: (i, 0)),
        out_shape=jax.ShapeDtypeStruct((NPAD, N_CLASSES), jnp.float32),
    )(a2p, g2, degp2, b2r)


def kernel(x, edge_index, W1, b1, W2, b2):
    ei = edge_index.astype(jnp.int32)
    pad = jnp.full((EPAD - E,), N, jnp.int32)
    src2 = jnp.concatenate([ei[0], pad]).reshape(EROWS, 128)
    dst2 = jnp.concatenate([ei[1], pad]).reshape(EROWS, 128)

    zeros_1d = jnp.zeros((NODES_PER_TILE,), jnp.float32)
    zeros_64 = jnp.zeros((1024, 64), jnp.bfloat16)
    zeros_16 = jnp.zeros((1024, N_CLASSES), jnp.float32)
    b1r = b1.reshape(1, D_HID)
    b2r = b2.reshape(1, N_CLASSES)

    degp = _deg_kernel(zeros_1d, dst2)
    degp2 = degp.reshape(2, NPAD, 1)
    g1 = _g1_tc(x, W1, degp2)
    a1 = _agg_128(zeros_64, src2, dst2, g1)
    g2 = _g2_tc(a1, g1, degp2, b1r, W2)
    a2p = _agg_16(zeros_16, src2, dst2, g2)
    out = _out_tc(a2p, g2, degp2, b2r)
    return out[:N]


# speedup vs baseline: 41.3230x; 1.0706x over previous
import functools

import jax
import jax.numpy as jnp
from jax import lax
from jax.experimental import pallas as pl
from jax.experimental.pallas import tpu as pltpu
from jax.experimental.pallas import tpu_sc as plsc

N = 10000
E = 320000
D_IN = 128
D_HID = 128
N_CLASSES = 16

NPAD = 10240
EPAD = 327680
EROWS = EPAD // 128
ROWS_PER_TILE = EROWS // 32
CHUNK_ROWS = 4
N_STEPS = ROWS_PER_TILE // CHUNK_ROWS
NODES_PER_TILE = NPAD // 16

_MESH = plsc.VectorSubcoreMesh(core_axis_name="c", subcore_axis_name="s")


@functools.partial(
    pl.kernel,
    mesh=_MESH,
    compiler_params=pltpu.CompilerParams(use_tc_tiling_on_sc=False),
    out_type=jax.ShapeDtypeStruct((2, NPAD), jnp.float32),
    scratch_types=[
        pltpu.VMEM((ROWS_PER_TILE, 128), jnp.int32),
        pltpu.VMEM((128,), jnp.float32),
        pltpu.VMEM_SHARED((NPAD,), jnp.float32),
        pltpu.SemaphoreType.DMA,
        pltpu.SemaphoreType.DMA,
    ],
)
def _deg_kernel(zeros_hbm, dst_hbm, out_hbm, didx, ones, acc, ssem0, ssem1):
    cid = lax.axis_index("c")
    sid = lax.axis_index("s")
    for j in range(8):
        ones[pl.ds(j * 16, 16)] = jnp.ones((16,), jnp.float32)
    base_n = sid * NODES_PER_TILE
    row0 = (cid * 16 + sid) * ROWS_PER_TILE
    pltpu.sync_copy(dst_hbm.at[pl.ds(row0, ROWS_PER_TILE)], didx)
    pltpu.sync_copy(zeros_hbm, acc.at[pl.ds(base_n, NODES_PER_TILE)])
    plsc.subcore_barrier()

    def fire(c, sem):
        for j in range(CHUNK_ROWS):
            pltpu.async_copy(ones, acc.at[didx.at[c * CHUNK_ROWS + j]], sem,
                             add=True)

    dummy = zeros_hbm.at[pl.ds(0, 128)]

    def drain(sem):
        for _ in range(CHUNK_ROWS):
            pltpu.make_async_copy(dummy, ones, sem).wait()

    fire(0, ssem0)

    def body(it, carry):
        a = 2 * it
        fire(a + 1, ssem1)
        drain(ssem0)
        fire(a + 2, ssem0)
        drain(ssem1)
        return carry

    lax.fori_loop(0, N_STEPS // 2 - 1, body, 0)
    fire(N_STEPS - 1, ssem1)
    drain(ssem0)
    drain(ssem1)
    plsc.subcore_barrier()
    pltpu.sync_copy(acc.at[pl.ds(base_n, NODES_PER_TILE)],
                    out_hbm.at[cid, pl.ds(base_n, NODES_PER_TILE)])


_CHUNK_E = CHUNK_ROWS * 128
_TAIL = NODES_PER_TILE - _CHUNK_E


def _make_agg(d, split_edges, macro_chunks, dtype=jnp.float32,
              chunk_rows=CHUNK_ROWS, table_spmem=False, table_per_core=False,
              stage_col_split=False):
    rows_per_tile = EROWS // 32 if split_edges else EROWS // 16
    steps = rows_per_tile // chunk_rows
    mc = macro_chunks
    mrows = mc * chunk_rows
    pairs = steps // (2 * mc)
    assert steps == pairs * 2 * mc

    @functools.partial(
        pl.kernel,
        mesh=_MESH,
        compiler_params=pltpu.CompilerParams(use_tc_tiling_on_sc=False),
        out_type=jax.ShapeDtypeStruct((2, NPAD, d), dtype),
        scratch_types=[
            pltpu.VMEM((mrows, 128), jnp.int32),
            pltpu.VMEM((mrows, 128), jnp.int32),
            pltpu.VMEM((mrows, 128), jnp.int32),
            pltpu.VMEM((mrows, 128), jnp.int32),
            pltpu.VMEM((chunk_rows * 128, d), dtype),
            pltpu.VMEM((chunk_rows * 128, d), dtype),
            pltpu.VMEM_SHARED((NPAD, d), dtype),
            (pltpu.VMEM_SHARED((NPAD, d), dtype)
             if table_spmem else pltpu.SMEM((1,), jnp.int32)),
            pltpu.SemaphoreType.DMA,
            pltpu.SemaphoreType.DMA,
            pltpu.SemaphoreType.DMA,
            pltpu.SemaphoreType.DMA,
        ],
    )
    def agg_kernel(zeros_hbm, src_hbm, dst_hbm, g_hbm, out_hbm,
                   sidxA, didxA, sidxB, didxB, rows0, rows1, acc, tbl,
                   gsem0, gsem1, ssem0, ssem1):
        cid = lax.axis_index("c")
        sid = lax.axis_index("s")
        base_n = sid * NODES_PER_TILE
        if split_edges:
            row0 = (cid * 16 + sid) * rows_per_tile
        else:
            row0 = sid * rows_per_tile
        src_table = g_hbm.at[cid] if table_per_core else g_hbm
        table = tbl if table_spmem else src_table

        rowss = [rows0, rows1]
        gsems = [gsem0, gsem1]
        ssems = [ssem0, ssem1]
        chunk_e = chunk_rows * 128
        dummy = zeros_hbm.at[pl.ds(0, chunk_e)]

        def load_idx(sbuf, dbuf, r):
            pltpu.sync_copy(src_hbm.at[pl.ds(r, mrows)], sbuf)
            pltpu.sync_copy(dst_hbm.at[pl.ds(r, mrows)], dbuf)

        def fire_g(sbuf, loc, buf, sem):
            for j in range(chunk_rows):
                pltpu.async_copy(table.at[sbuf.at[loc * chunk_rows + j]],
                                 buf.at[pl.ds(j * 128, 128)], sem)

        def fire_s(dbuf, loc, buf, sem):
            for j in range(chunk_rows):
                pltpu.async_copy(buf.at[pl.ds(j * 128, 128)],
                                 acc.at[dbuf.at[loc * chunk_rows + j]], sem,
                                 add=True)

        def drain(sem, buf):
            pltpu.make_async_copy(dummy, buf, sem).wait()

        def pair_body(u, reload_a, fire_next):
            for i in range(2 * mc):
                p = i % 2
                in_a = i < mc
                loc = i if in_a else i - mc
                drain(gsems[p], rowss[p])
                fire_s(didxA if in_a else didxB, loc, rowss[p], ssems[p])
                drain(ssems[1 - p], rowss[1 - p])
                if i == 0:
                    load_idx(sidxB, didxB, row0 + (2 * u + 1) * mrows)
                if i == mc and reload_a:
                    load_idx(sidxA, didxA, row0 + (2 * u + 2) * mrows)
                if i < 2 * mc - 1:
                    i1 = i + 1
                    sb = sidxA if i1 < mc else sidxB
                    l1 = i1 if i1 < mc else i1 - mc
                    fire_g(sb, l1, rowss[1 - p], gsems[1 - p])
                elif fire_next:
                    fire_g(sidxA, 0, rowss[1 - p], gsems[1 - p])

        load_idx(sidxA, didxA, row0)
        pltpu.sync_copy(zeros_hbm.at[pl.ds(0, NODES_PER_TILE)],
                        acc.at[pl.ds(base_n, NODES_PER_TILE)])
        if table_spmem and stage_col_split:
            pltpu.sync_copy(
                g_hbm.at[pl.ds(base_n, NODES_PER_TILE), pl.ds(cid * d, d)],
                tbl.at[pl.ds(base_n, NODES_PER_TILE)])
        elif table_spmem:
            pltpu.sync_copy(src_table.at[pl.ds(base_n, NODES_PER_TILE)],
                            tbl.at[pl.ds(base_n, NODES_PER_TILE)])
        pltpu.sync_copy(dummy, rows1)
        plsc.subcore_barrier()
        fire_g(sidxA, 0, rows0, gsem0)
        fire_s(didxA, 0, rows1, ssem1)

        def body(u, carry):
            pair_body(u, reload_a=True, fire_next=True)
            return carry

        lax.fori_loop(0, pairs - 1, body, 0)
        pair_body(pairs - 1, reload_a=False, fire_next=False)
        drain(ssems[1], rowss[1])
        plsc.subcore_barrier()
        pltpu.sync_copy(acc.at[pl.ds(base_n, NODES_PER_TILE)],
                        out_hbm.at[cid, pl.ds(base_n, NODES_PER_TILE)])

    return agg_kernel


_agg_128 = _make_agg(64, split_edges=False, macro_chunks=2,
                     dtype=jnp.bfloat16, chunk_rows=8, table_spmem=True,
                     stage_col_split=True)
_agg_16 = _make_agg(N_CLASSES, split_edges=True, macro_chunks=5, chunk_rows=8,
                    table_spmem=True)


RB = 1024
GRID = NPAD // RB


def _dinv_block(degp_ref):
    d = jnp.transpose(degp_ref[...])
    return lax.rsqrt(d[:, 0:1] + d[:, 1:2] + 1.0)


def _g1_body(x_ref, w_ref, degp_ref, g_ref):
    h = jnp.dot(x_ref[...], w_ref[...], preferred_element_type=jnp.float32)
    g_ref[...] = (h * _dinv_block(degp_ref)).astype(jnp.bfloat16)


def _g1_tc(x_pad, w1, degp2):
    return pl.pallas_call(
        _g1_body,
        grid=(GRID,),
        in_specs=[
            pl.BlockSpec((RB, D_IN), lambda i: (i, 0)),
            pl.BlockSpec((D_IN, D_HID), lambda i: (0, 0)),
            pl.BlockSpec((2, RB), lambda i: (0, i)),
        ],
        out_specs=pl.BlockSpec((RB, D_HID), lambda i: (i, 0)),
        out_shape=jax.ShapeDtypeStruct((NPAD, D_HID), jnp.bfloat16),
    )(x_pad, w1, degp2)


def _g2_body(a1_ref, g1_ref, degp_ref, b1_ref, w2_ref, g2_ref):
    dinv = _dinv_block(degp_ref)
    a = a1_ref[...].astype(jnp.float32)
    sfull = (jnp.concatenate([a[0], a[1]], axis=1)
             + g1_ref[...].astype(jnp.float32))
    pre = sfull * dinv + b1_ref[...]
    h = jnp.maximum(pre, 0.0)
    g2_ref[...] = (
        jnp.dot(h, w2_ref[...], preferred_element_type=jnp.float32)
        * dinv)


def _g2_tc(a1, g1, degp2, b1r, w2):
    return pl.pallas_call(
        _g2_body,
        grid=(GRID,),
        in_specs=[
            pl.BlockSpec((2, RB, 64), lambda i: (0, i, 0)),
            pl.BlockSpec((RB, D_HID), lambda i: (i, 0)),
            pl.BlockSpec((2, RB), lambda i: (0, i)),
            pl.BlockSpec((1, D_HID), lambda i: (0, 0)),
            pl.BlockSpec((D_HID, N_CLASSES), lambda i: (0, 0)),
        ],
        out_specs=pl.BlockSpec((RB, N_CLASSES), lambda i: (i, 0)),
        out_shape=jax.ShapeDtypeStruct((NPAD, N_CLASSES), jnp.float32),
    )(a1, g1, degp2, b1r, w2)


def _out_body(a2_ref, g2_ref, degp_ref, b2_ref, o_ref):
    a = a2_ref[...]
    s = a[0] + a[1] + g2_ref[...]
    o_ref[...] = s * _dinv_block(degp_ref) + b2_ref[...]


def _out_tc(a2p, g2, degp2, b2r):
    return pl.pallas_call(
        _out_body,
        grid=(GRID,),
        in_specs=[
            pl.BlockSpec((2, RB, N_CLASSES), lambda i: (0, i, 0)),
            pl.BlockSpec((RB, N_CLASSES), lambda i: (i, 0)),
            pl.BlockSpec((2, RB), lambda i: (0, i)),
            pl.BlockSpec((1, N_CLASSES), lambda i: (0, 0)),
        ],
        out_specs=pl.BlockSpec((RB, N_CLASSES), lambda i: (i, 0)),
        out_shape=jax.ShapeDtypeStruct((NPAD, N_CLASSES), jnp.float32),
    )(a2p, g2, degp2, b2r)


def kernel(x, edge_index, W1, b1, W2, b2):
    ei = edge_index.astype(jnp.int32)
    pad = jnp.full((EPAD - E,), N, jnp.int32)
    src2 = jnp.concatenate([ei[0], pad]).reshape(EROWS, 128)
    dst2 = jnp.concatenate([ei[1], pad]).reshape(EROWS, 128)

    zeros_1d = jnp.zeros((NODES_PER_TILE,), jnp.float32)
    zeros_64 = jnp.zeros((1024, 64), jnp.bfloat16)
    zeros_16 = jnp.zeros((1024, N_CLASSES), jnp.float32)
    b1r = b1.reshape(1, D_HID)
    b2r = b2.reshape(1, N_CLASSES)

    degp2 = _deg_kernel(zeros_1d, dst2)
    g1 = _g1_tc(x, W1, degp2)
    a1 = _agg_128(zeros_64, src2, dst2, g1)
    g2 = _g2_tc(a1, g1, degp2, b1r, W2)
    a2p = _agg_16(zeros_16, src2, dst2, g2)
    out = _out_tc(a2p, g2, degp2, b2r)
    return out[:N]


# speedup vs baseline: 41.9117x; 1.0142x over previous
import functools

import jax
import jax.numpy as jnp
from jax import lax
from jax.experimental import pallas as pl
from jax.experimental.pallas import tpu as pltpu
from jax.experimental.pallas import tpu_sc as plsc

N = 10000
E = 320000
D_IN = 128
D_HID = 128
N_CLASSES = 16

NPAD = 10240
EPAD = 327680
EROWS = EPAD // 128
ROWS_PER_TILE = EROWS // 32
CHUNK_ROWS = 4
N_STEPS = ROWS_PER_TILE // CHUNK_ROWS
NODES_PER_TILE = NPAD // 16

_MESH = plsc.VectorSubcoreMesh(core_axis_name="c", subcore_axis_name="s")


@functools.partial(
    pl.kernel,
    mesh=_MESH,
    compiler_params=pltpu.CompilerParams(use_tc_tiling_on_sc=False),
    out_type=jax.ShapeDtypeStruct((2, NPAD), jnp.float32),
    scratch_types=[
        pltpu.VMEM((ROWS_PER_TILE, 128), jnp.int32),
        pltpu.VMEM((128,), jnp.float32),
        pltpu.VMEM_SHARED((NPAD,), jnp.float32),
        pltpu.SemaphoreType.DMA,
        pltpu.SemaphoreType.DMA,
    ],
)
def _deg_kernel(zeros_hbm, dst_hbm, out_hbm, didx, ones, acc, ssem0, ssem1):
    cid = lax.axis_index("c")
    sid = lax.axis_index("s")
    for j in range(8):
        ones[pl.ds(j * 16, 16)] = jnp.ones((16,), jnp.float32)
    base_n = sid * NODES_PER_TILE
    row0 = (cid * 16 + sid) * ROWS_PER_TILE
    pltpu.sync_copy(dst_hbm.at[pl.ds(row0, ROWS_PER_TILE)], didx)
    pltpu.sync_copy(zeros_hbm, acc.at[pl.ds(base_n, NODES_PER_TILE)])
    plsc.subcore_barrier()

    def fire(c, sem):
        for j in range(CHUNK_ROWS):
            pltpu.async_copy(ones, acc.at[didx.at[c * CHUNK_ROWS + j]], sem,
                             add=True)

    dummy = zeros_hbm.at[pl.ds(0, 128)]

    def drain(sem):
        for _ in range(CHUNK_ROWS):
            pltpu.make_async_copy(dummy, ones, sem).wait()

    fire(0, ssem0)

    def body(it, carry):
        a = 2 * it
        fire(a + 1, ssem1)
        drain(ssem0)
        fire(a + 2, ssem0)
        drain(ssem1)
        return carry

    lax.fori_loop(0, N_STEPS // 2 - 1, body, 0)
    fire(N_STEPS - 1, ssem1)
    drain(ssem0)
    drain(ssem1)
    plsc.subcore_barrier()
    pltpu.sync_copy(acc.at[pl.ds(base_n, NODES_PER_TILE)],
                    out_hbm.at[cid, pl.ds(base_n, NODES_PER_TILE)])


_CHUNK_E = CHUNK_ROWS * 128
_TAIL = NODES_PER_TILE - _CHUNK_E


def _make_agg(d, split_edges, macro_chunks, dtype=jnp.float32,
              chunk_rows=CHUNK_ROWS, table_spmem=False, table_per_core=False,
              stage_col_split=False, stage_first_cols=False):
    rows_per_tile = EROWS // 32 if split_edges else EROWS // 16
    steps = rows_per_tile // chunk_rows
    mc = macro_chunks
    mrows = mc * chunk_rows
    pairs = steps // (2 * mc)
    assert steps == pairs * 2 * mc

    @functools.partial(
        pl.kernel,
        mesh=_MESH,
        compiler_params=pltpu.CompilerParams(use_tc_tiling_on_sc=False),
        out_type=jax.ShapeDtypeStruct((2, NPAD, d), dtype),
        scratch_types=[
            pltpu.VMEM((mrows, 128), jnp.int32),
            pltpu.VMEM((mrows, 128), jnp.int32),
            pltpu.VMEM((mrows, 128), jnp.int32),
            pltpu.VMEM((mrows, 128), jnp.int32),
            pltpu.VMEM((chunk_rows * 128, d), dtype),
            pltpu.VMEM((chunk_rows * 128, d), dtype),
            pltpu.VMEM_SHARED((NPAD, d), dtype),
            (pltpu.VMEM_SHARED((NPAD, d), dtype)
             if table_spmem else pltpu.SMEM((1,), jnp.int32)),
            pltpu.SemaphoreType.DMA,
            pltpu.SemaphoreType.DMA,
            pltpu.SemaphoreType.DMA,
            pltpu.SemaphoreType.DMA,
        ],
    )
    def agg_kernel(zeros_hbm, src_hbm, dst_hbm, g_hbm, out_hbm,
                   sidxA, didxA, sidxB, didxB, rows0, rows1, acc, tbl,
                   gsem0, gsem1, ssem0, ssem1):
        cid = lax.axis_index("c")
        sid = lax.axis_index("s")
        base_n = sid * NODES_PER_TILE
        if split_edges:
            row0 = (cid * 16 + sid) * rows_per_tile
        else:
            row0 = sid * rows_per_tile
        src_table = g_hbm.at[cid] if table_per_core else g_hbm
        table = tbl if table_spmem else src_table

        rowss = [rows0, rows1]
        gsems = [gsem0, gsem1]
        ssems = [ssem0, ssem1]
        chunk_e = chunk_rows * 128
        dummy = zeros_hbm.at[pl.ds(0, chunk_e)]

        def load_idx(sbuf, dbuf, r):
            pltpu.sync_copy(src_hbm.at[pl.ds(r, mrows)], sbuf)
            pltpu.sync_copy(dst_hbm.at[pl.ds(r, mrows)], dbuf)

        def fire_g(sbuf, loc, buf, sem):
            for j in range(chunk_rows):
                pltpu.async_copy(table.at[sbuf.at[loc * chunk_rows + j]],
                                 buf.at[pl.ds(j * 128, 128)], sem)

        def fire_s(dbuf, loc, buf, sem):
            for j in range(chunk_rows):
                pltpu.async_copy(buf.at[pl.ds(j * 128, 128)],
                                 acc.at[dbuf.at[loc * chunk_rows + j]], sem,
                                 add=True)

        def drain(sem, buf):
            pltpu.make_async_copy(dummy, buf, sem).wait()

        def pair_body(u, reload_a, fire_next):
            for i in range(2 * mc):
                p = i % 2
                in_a = i < mc
                loc = i if in_a else i - mc
                drain(gsems[p], rowss[p])
                fire_s(didxA if in_a else didxB, loc, rowss[p], ssems[p])
                drain(ssems[1 - p], rowss[1 - p])
                if i == 0:
                    load_idx(sidxB, didxB, row0 + (2 * u + 1) * mrows)
                if i == mc and reload_a:
                    load_idx(sidxA, didxA, row0 + (2 * u + 2) * mrows)
                if i < 2 * mc - 1:
                    i1 = i + 1
                    sb = sidxA if i1 < mc else sidxB
                    l1 = i1 if i1 < mc else i1 - mc
                    fire_g(sb, l1, rowss[1 - p], gsems[1 - p])
                elif fire_next:
                    fire_g(sidxA, 0, rowss[1 - p], gsems[1 - p])

        load_idx(sidxA, didxA, row0)
        pltpu.sync_copy(zeros_hbm.at[pl.ds(0, NODES_PER_TILE)],
                        acc.at[pl.ds(base_n, NODES_PER_TILE)])
        if table_spmem and stage_first_cols:
            pltpu.sync_copy(
                g_hbm.at[pl.ds(base_n, NODES_PER_TILE), pl.ds(0, d)],
                tbl.at[pl.ds(base_n, NODES_PER_TILE)])
        elif table_spmem and stage_col_split:
            pltpu.sync_copy(
                g_hbm.at[pl.ds(base_n, NODES_PER_TILE), pl.ds(cid * d, d)],
                tbl.at[pl.ds(base_n, NODES_PER_TILE)])
        elif table_spmem:
            pltpu.sync_copy(src_table.at[pl.ds(base_n, NODES_PER_TILE)],
                            tbl.at[pl.ds(base_n, NODES_PER_TILE)])
        pltpu.sync_copy(dummy, rows1)
        plsc.subcore_barrier()
        fire_g(sidxA, 0, rows0, gsem0)
        fire_s(didxA, 0, rows1, ssem1)

        def body(u, carry):
            pair_body(u, reload_a=True, fire_next=True)
            return carry

        lax.fori_loop(0, pairs - 1, body, 0)
        pair_body(pairs - 1, reload_a=False, fire_next=False)
        drain(ssems[1], rowss[1])
        plsc.subcore_barrier()
        pltpu.sync_copy(acc.at[pl.ds(base_n, NODES_PER_TILE)],
                        out_hbm.at[cid, pl.ds(base_n, NODES_PER_TILE)])

    return agg_kernel


_agg_128 = _make_agg(64, split_edges=False, macro_chunks=2,
                     dtype=jnp.bfloat16, chunk_rows=8, table_spmem=True,
                     stage_col_split=True)
_agg_16 = _make_agg(N_CLASSES, split_edges=True, macro_chunks=5, chunk_rows=8,
                    table_spmem=True, stage_first_cols=True)


RB = 1024
GRID = NPAD // RB


def _dinv_block(degp_ref):
    d = jnp.transpose(degp_ref[...])
    return lax.rsqrt(d[:, 0:1] + d[:, 1:2] + 1.0)


def _g1_body(x_ref, w_ref, degp_ref, g_ref):
    h = jnp.dot(x_ref[...], w_ref[...], preferred_element_type=jnp.float32)
    g_ref[...] = (h * _dinv_block(degp_ref)).astype(jnp.bfloat16)


def _g1_tc(x_pad, w1, degp2):
    return pl.pallas_call(
        _g1_body,
        grid=(GRID,),
        in_specs=[
            pl.BlockSpec((RB, D_IN), lambda i: (i, 0)),
            pl.BlockSpec((D_IN, D_HID), lambda i: (0, 0)),
            pl.BlockSpec((2, RB), lambda i: (0, i)),
        ],
        out_specs=pl.BlockSpec((RB, D_HID), lambda i: (i, 0)),
        out_shape=jax.ShapeDtypeStruct((NPAD, D_HID), jnp.bfloat16),
    )(x_pad, w1, degp2)


def _g2_body(a1_ref, g1_ref, degp_ref, b1_ref, w2_ref, g2_ref):
    dinv = _dinv_block(degp_ref)
    a = a1_ref[...].astype(jnp.float32)
    sfull = (jnp.concatenate([a[0], a[1]], axis=1)
             + g1_ref[...].astype(jnp.float32))
    pre = sfull * dinv + b1_ref[...]
    h = jnp.maximum(pre, 0.0)
    g2 = (jnp.dot(h, w2_ref[...], preferred_element_type=jnp.float32)
          * dinv)
    g2_ref[...] = jnp.concatenate(
        [g2, jnp.zeros((g2.shape[0], 128 - N_CLASSES), jnp.float32)], axis=1)


def _g2_tc(a1, g1, degp2, b1r, w2):
    return pl.pallas_call(
        _g2_body,
        grid=(GRID,),
        in_specs=[
            pl.BlockSpec((2, RB, 64), lambda i: (0, i, 0)),
            pl.BlockSpec((RB, D_HID), lambda i: (i, 0)),
            pl.BlockSpec((2, RB), lambda i: (0, i)),
            pl.BlockSpec((1, D_HID), lambda i: (0, 0)),
            pl.BlockSpec((D_HID, N_CLASSES), lambda i: (0, 0)),
        ],
        out_specs=pl.BlockSpec((RB, 128), lambda i: (i, 0)),
        out_shape=jax.ShapeDtypeStruct((NPAD, 128), jnp.float32),
    )(a1, g1, degp2, b1r, w2)


def _out_body(a2_ref, g2_ref, degp_ref, b2_ref, o_ref):
    a = a2_ref[...]
    s = a[0] + a[1] + g2_ref[...][:, :N_CLASSES]
    o_ref[...] = s * _dinv_block(degp_ref) + b2_ref[...]


def _out_tc(a2p, g2, degp2, b2r):
    return pl.pallas_call(
        _out_body,
        grid=(GRID,),
        in_specs=[
            pl.BlockSpec((2, RB, N_CLASSES), lambda i: (0, i, 0)),
            pl.BlockSpec((RB, 128), lambda i: (i, 0)),
            pl.BlockSpec((2, RB), lambda i: (0, i)),
            pl.BlockSpec((1, N_CLASSES), lambda i: (0, 0)),
        ],
        out_specs=pl.BlockSpec((RB, N_CLASSES), lambda i: (i, 0)),
        out_shape=jax.ShapeDtypeStruct((NPAD, N_CLASSES), jnp.float32),
    )(a2p, g2, degp2, b2r)


def kernel(x, edge_index, W1, b1, W2, b2):
    ei = edge_index.astype(jnp.int32)
    pad = jnp.full((EPAD - E,), N, jnp.int32)
    src2 = jnp.concatenate([ei[0], pad]).reshape(EROWS, 128)
    dst2 = jnp.concatenate([ei[1], pad]).reshape(EROWS, 128)

    zeros_1d = jnp.zeros((NODES_PER_TILE,), jnp.float32)
    zeros_64 = jnp.zeros((1024, 64), jnp.bfloat16)
    zeros_16 = jnp.zeros((1024, N_CLASSES), jnp.float32)
    b1r = b1.reshape(1, D_HID)
    b2r = b2.reshape(1, N_CLASSES)

    degp2 = _deg_kernel(zeros_1d, dst2)
    g1 = _g1_tc(x, W1, degp2)
    a1 = _agg_128(zeros_64, src2, dst2, g1)
    g2 = _g2_tc(a1, g1, degp2, b1r, W2)
    a2p = _agg_16(zeros_16, src2, dst2, g2)
    out = _out_tc(a2p, g2, degp2, b2r)
    return out[:N]
